# baseline (speedup 1.0000x reference)
import jax
import jax.numpy as jnp
from jax.experimental import pallas as pl
from jax.experimental.pallas import tpu as pltpu

S, F, H = 50, 128, 32
BB = 256


P = 8
SP = -(-S // P)


def _decode_block(item_ref, g_ref, encW1_ref, encb1_ref, encW2_ref, encb2_ref,
                  Wx_ref, Wh_ref, lb_ref, dW1e_ref, dW1h4_ref, db1t4_ref,
                  dW2d_ref, db2t4_ref, w3d_ref, db3_ref,
                  probs_all_ref, probs_one_ref, idx_ref, scores_ref):
    item2 = item_ref[...].reshape(S * BB, F)

    e1 = jnp.maximum(
        jnp.dot(item2, encW1_ref[...], preferred_element_type=jnp.float32)
        + encb1_ref[...], 0.0)
    e2 = jnp.maximum(
        jnp.dot(e1, encW2_ref[...], preferred_element_type=jnp.float32)
        + encb2_ref[...], 0.0)
    e2p = jnp.concatenate(
        [e2.reshape(S, BB, 16), jnp.zeros((P * SP - S, BB, 16), jnp.float32)],
        axis=0).reshape(SP, P, BB, 16)
    parts = [jnp.dot(e2p[:, q].reshape(SP * BB, 16), dW1e_ref[...],
                     preferred_element_type=jnp.float32) for q in range(P)]
    encprojW = (jnp.concatenate(parts, axis=1) + db1t4_ref[...]).reshape(SP, BB, P * 32)

    itemx = jnp.dot(item2, Wx_ref[...],
                    preferred_element_type=jnp.float32).reshape(S, BB, F)

    iota_sb = jax.lax.broadcasted_iota(jnp.int32, (S, BB), 0)
    iota_l = jax.lax.broadcasted_iota(jnp.int32, (S, BB, 1), 0)

    fg_one = jnp.where(
        (jax.lax.broadcasted_iota(jnp.int32, (1, 4 * H), 1) // H) == 2, 1.0, 0.0)

    def body(k, carry):
        h, c, mb, gx, p1, ia, sa = carry
        gates = (gx
                 + jnp.dot(h, Wh_ref[...], preferred_element_type=jnp.float32)
                 + lb_ref[...])
        sg = jax.nn.sigmoid(gates + fg_one)
        c = (c * sg[:, 2 * H:3 * H]
             + sg[:, 0:H] * jnp.tanh(gates[:, H:2 * H]))
        h = jnp.tanh(c) * sg[:, 3 * H:4 * H]

        hp4 = jnp.dot(h, dW1h4_ref[...], preferred_element_type=jnp.float32)
        z1 = jnp.maximum(encprojW + hp4[None, :, :], 0.0)
        z2 = jnp.maximum(
            jnp.dot(z1.reshape(SP * BB, P * 32), dW2d_ref[...],
                    preferred_element_type=jnp.float32) + db2t4_ref[...], 0.0)
        lw = jnp.dot(z2, w3d_ref[...],
                     preferred_element_type=jnp.float32)
        lt = jnp.transpose(lw.reshape(SP, BB, P), (0, 2, 1))
        logits = lt.reshape(P * SP, BB)[:S]
        logits = logits + db3_ref[...]

        ml = mb + logits
        m = jnp.max(ml, axis=0, keepdims=True)
        e = jnp.exp(ml - m)
        p = e / jnp.sum(e, axis=0, keepdims=True)

        y = jnp.log(p + 1e-20) + g_ref[pl.ds(k, 1)][0]
        mx = jnp.max(y, axis=0, keepdims=True)
        idxv = jnp.min(jnp.where(y == mx, iota_sb, S), axis=0, keepdims=True)
        oh = (iota_sb == idxv).astype(jnp.float32)

        mb = mb + oh * (-1e9)
        p1 = jnp.where(iota_sb == k, jnp.sum(p * oh, axis=0, keepdims=True), p1)
        ia = jnp.where(iota_sb == k, idxv, ia)
        sa = sa + oh * (1.0 - 0.03 * k.astype(jnp.float32))

        probs_all_ref[pl.ds(k, 1)] = p[None]

        idx_col = jnp.transpose(idxv)
        sel = iota_l == idx_col[None]
        gx = jnp.sum(jnp.where(sel, itemx, 0.0), axis=0)
        return h, c, mb, gx, p1, ia, sa

    zf = jnp.zeros((BB, H), jnp.float32)
    carry = (zf, zf, jnp.zeros((S, BB), jnp.float32), jnp.zeros((BB, F), jnp.float32),
             jnp.zeros((S, BB), jnp.float32), jnp.zeros((S, BB), jnp.int32),
             jnp.zeros((S, BB), jnp.float32))
    h, c, mb, gx, p1, ia, sa = jax.lax.fori_loop(0, S, body, carry, unroll=5)
    probs_one_ref[...] = p1
    idx_ref[...] = ia
    scores_ref[...] = sa


def kernel(item_input, enc_W1, enc_b1, enc_W2, enc_b2, lstm_kernel, lstm_bias,
           dec_W1, dec_b1, dec_W2, dec_b2, dec_W3, dec_b3):
    b = item_input.shape[0]
    nb = b // BB

    base = jax.random.key(42)
    g = jax.vmap(lambda k: jax.random.gumbel(jax.random.fold_in(base, k), (b, S),
                                             jnp.float32))(jnp.arange(S))
    g_t = jnp.transpose(g, (0, 2, 1))
    item_t = jnp.transpose(item_input, (1, 0, 2))

    Wx = lstm_kernel[:F]
    Wh = lstm_kernel[F:]
    dW1e = dec_W1[:16]
    dW1h4 = jnp.tile(dec_W1[16:], (1, P))
    db1t4 = jnp.tile(dec_b1, P).reshape(1, P * 32)
    dW2d = jnp.kron(jnp.eye(P, dtype=jnp.float32), dec_W2)
    db2t4 = jnp.tile(dec_b2, P).reshape(1, P * 16)
    w3d = jnp.kron(jnp.eye(P, dtype=jnp.float32), dec_W3)

    full = lambda shp: pl.BlockSpec(shp, lambda i: tuple(0 for _ in shp))
    in_specs = [
        pl.BlockSpec((S, BB, F), lambda i: (0, i, 0)),
        pl.BlockSpec((S, S, BB), lambda i: (0, 0, i)),
        full((F, 32)), full((1, 32)), full((32, 16)), full((1, 16)),
        full((F, 4 * H)), full((H, 4 * H)), full((1, 4 * H)),
        full((16, 32)), full((H, P * 32)), full((1, P * 32)),
        full((P * 32, P * 16)), full((1, P * 16)), full((P * 16, P)), full((1, 1)),
    ]
    out_specs = [
        pl.BlockSpec((S, S, BB), lambda i: (0, 0, i)),
        pl.BlockSpec((S, BB), lambda i: (0, i)),
        pl.BlockSpec((S, BB), lambda i: (0, i)),
        pl.BlockSpec((S, BB), lambda i: (0, i)),
    ]
    out_shape = [
        jax.ShapeDtypeStruct((S, S, b), jnp.float32),
        jax.ShapeDtypeStruct((S, b), jnp.float32),
        jax.ShapeDtypeStruct((S, b), jnp.int32),
        jax.ShapeDtypeStruct((S, b), jnp.float32),
    ]
    pa_t, p1_t, idx_t, sc_t = pl.pallas_call(
        _decode_block,
        grid=(nb,),
        in_specs=in_specs,
        out_specs=out_specs,
        out_shape=out_shape,
        compiler_params=pltpu.CompilerParams(
            dimension_semantics=("parallel",)),
    )(item_t, g_t, enc_W1, enc_b1.reshape(1, 32), enc_W2, enc_b2.reshape(1, 16),
      Wx, Wh, lstm_bias.reshape(1, 4 * H), dW1e, dW1h4, db1t4,
      dW2d, db2t4, w3d, dec_b3.reshape(1, 1))
    probs_all = jnp.transpose(pa_t, (2, 0, 1))
    probs_one = jnp.transpose(p1_t)
    idx = jnp.transpose(idx_t)
    scores = jnp.transpose(sc_t)
    return (probs_all, probs_one, idx, scores.reshape(-1, 1))
